# baseline (speedup 1.0000x reference)
import dataclasses
import functools

import jax
import jax.numpy as jnp
import numpy as np
from jax import lax
from jax.experimental import pallas as pl
from jax.experimental.pallas import tpu as pltpu
from jax.experimental.pallas import tpu_sc as plsc

_N = 10000
_E = 320000
_D = 128
_H = 512
_NC = 2
_NS = 16
_NW = _NC * _NS
_EPW = _E // _NW
_K = 80
_CHUNKS = _EPW // _K
_RPT = 624
_ZR = 16
_SPLAT_IDX = [np.full((16,), i, np.int32) for i in range(16)]


def _make_seg_sum_128():
  mesh = plsc.VectorSubcoreMesh(core_axis_name="c", subcore_axis_name="s")
  jper = _D // 16
  width = _D
  idx_t = [pltpu.VMEM((_K,), jnp.int32)] * 4
  ew_t = [pltpu.VMEM((_K,), jnp.float32)] * 4
  row_t = [pltpu.VMEM((_K, width), jnp.float32)] * 4
  sem_t = [pltpu.SemaphoreType.DMA] * 8

  @functools.partial(
      pl.kernel,
      out_type=jax.ShapeDtypeStruct((_NC, _N, width), jnp.float32),
      mesh=mesh,
      scratch_types=(idx_t + idx_t + ew_t + row_t +
                     [pltpu.VMEM_SHARED((_N, width), jnp.float32)] + sem_t),
      compiler_params=dataclasses.replace(
          pltpu.CompilerParams(), needs_layout_passes=False),
  )
  def seg_kernel(tab_hbm, src_hbm, dst_hbm, ew_hbm, out_hbm,
                 si0, si1, si2, si3, di0, di1, di2, di3, wi0, wi1, wi2, wi3,
                 gb0, gb1, sb0, sb1, acc_sh,
                 gsem0, gsem1, ssem0, ssem1, isem0, isem1, isem2, isem3):
    c = lax.axis_index("c")
    s = lax.axis_index("s")
    wid = c * _NS + s
    e0 = wid * _EPW

    src_ib = (si0, si1, si2, si3)
    dst_ib = (di0, di1, di2, di3)
    ew_ib = (wi0, wi1, wi2, wi3)
    gbufs = (gb0, gb1)
    sbufs = (sb0, sb1)
    gsems = (gsem0, gsem1)
    ssems = (ssem0, ssem1)
    isems = (isem0, isem1, isem2, isem3)

    zvec = jnp.zeros((16,), jnp.float32)

    @pl.loop(0, _ZR)
    def _(i):
      for j in range(jper):
        gb0[i, pl.ds(j * 16, 16)] = zvec

    nz = jnp.where(s == _NS - 1, (_N - (_NS - 1) * _RPT) // _ZR, _RPT // _ZR)

    @pl.loop(0, nz)
    def _(z):
      pltpu.sync_copy(gb0.at[pl.ds(0, _ZR)],
                      acc_sh.at[pl.ds(s * _RPT + z * _ZR, _ZR)])
    plsc.subcore_barrier()

    def fire_idx(cx, b4):
      base = e0 + cx * _K
      pltpu.async_copy(src_hbm.at[pl.ds(base, _K)], src_ib[b4], isems[b4])
      pltpu.async_copy(dst_hbm.at[pl.ds(base, _K)], dst_ib[b4], isems[b4])
      pltpu.async_copy(ew_hbm.at[pl.ds(base, _K)], ew_ib[b4], isems[b4])

    def wait_idx(cx, b4):
      base = e0 + cx * _K
      pltpu.make_async_copy(src_hbm.at[pl.ds(base, _K)], src_ib[b4],
                            isems[b4]).wait()
      pltpu.make_async_copy(dst_hbm.at[pl.ds(base, _K)], dst_ib[b4],
                            isems[b4]).wait()
      pltpu.make_async_copy(ew_hbm.at[pl.ds(base, _K)], ew_ib[b4],
                            isems[b4]).wait()

    def fire_gather(cc, b2, b4):
      pltpu.async_copy(tab_hbm.at[src_ib[b4]], gbufs[b2], gsems[b2])

    def wait_gather(b2, b4):
      pltpu.make_async_copy(tab_hbm.at[src_ib[b4]], gbufs[b2],
                            gsems[b2]).wait()

    def fire_scatter(b2, b4):
      pltpu.async_copy(sbufs[b2], acc_sh.at[dst_ib[b4]], ssems[b2], add=True)

    def wait_scatter(b2, b4):
      pltpu.make_async_copy(sbufs[b2], acc_sh.at[dst_ib[b4]],
                            ssems[b2]).wait()

    def slot(cc, b2, b4, steady_tail):
      wait_gather(b2, b4)

      @pl.when(cc >= 2)
      def _():
        wait_scatter(b2, (b4 + 2) % 4)

      if not steady_tail:
        @pl.when(cc + 2 < _CHUNKS)
        def _():
          fire_idx(cc + 2, (b4 + 2) % 4)

      for g in range(_K // 16):
        wvec = ew_ib[b4][pl.ds(g * 16, 16)]
        for e16 in range(16):
          w16 = lax.gather(
              wvec, jnp.full((16, 1), e16, jnp.int32),
              dimension_numbers=lax.GatherDimensionNumbers(
                  offset_dims=(), collapsed_slice_dims=(0,),
                  start_index_map=(0,)),
              slice_sizes=(1,),
              mode=lax.GatherScatterMode.PROMISE_IN_BOUNDS)
          e = g * 16 + e16
          for j in range(jper):
            sbufs[b2][e, pl.ds(j * 16, 16)] = (
                gbufs[b2][e, pl.ds(j * 16, 16)] * w16)

      fire_scatter(b2, b4)
      if not steady_tail:
        @pl.when(cc + 2 < _CHUNKS)
        def _():
          wait_idx(cc + 2, (b4 + 2) % 4)
          fire_gather(cc + 2, b2, (b4 + 2) % 4)

    fire_idx(0, 0)
    fire_idx(1, 1)
    wait_idx(0, 0)
    fire_gather(0, 0, 0)
    wait_idx(1, 1)
    fire_gather(1, 1, 1)

    @pl.loop(0, _CHUNKS - 1, step=4)
    def _(cc):
      slot(cc, 0, 0, False)
      slot(cc + 1, 1, 1, False)
      slot(cc + 2, 0, 2, False)
      slot(cc + 3, 1, 3, False)

    slot(_CHUNKS - 1, 0, 0, True)
    wait_scatter(1, 3)
    wait_scatter(0, 0)

    plsc.subcore_barrier()

    @pl.loop(0, nz)
    def _(z):
      r0 = s * _RPT + z * _ZR
      pltpu.sync_copy(acc_sh.at[pl.ds(r0, _ZR)], out_hbm.at[c, pl.ds(r0, _ZR)])

  return seg_kernel


_seg_sum_128 = _make_seg_sum_128()

_F = 2 * _N
_FPT = 1248
_NV = _EPW // 16


def _make_seg_sum_pair():
  mesh = plsc.VectorSubcoreMesh(core_axis_name="c", subcore_axis_name="s")

  @functools.partial(
      pl.kernel,
      out_type=jax.ShapeDtypeStruct((_NC * _F,), jnp.float32),
      mesh=mesh,
      scratch_types=[
          pltpu.VMEM((_F,), jnp.float32),
          pltpu.VMEM((_EPW,), jnp.int32),
          pltpu.VMEM((_EPW,), jnp.int32),
          pltpu.VMEM((_EPW,), jnp.float32),
          pltpu.VMEM((_F,), jnp.float32),
          pltpu.VMEM((1280,), jnp.float32),
          pltpu.VMEM((1280,), jnp.float32),
          pltpu.VMEM_SHARED((_NS * _F,), jnp.float32),
          pltpu.SemaphoreType.DMA,
      ],
      compiler_params=dataclasses.replace(
          pltpu.CompilerParams(), needs_layout_passes=False),
  )
  def pair_kernel(t01_hbm, src_hbm, dst_hbm, ew_hbm, out_hbm,
                  t01_v, src_v, dst_v, ew_v, acc_v, red_v, tmp_v, accs_sh,
                  sem):
    c = lax.axis_index("c")
    s = lax.axis_index("s")
    wid = c * _NS + s
    e0 = wid * _EPW

    pltpu.sync_copy(t01_hbm, t01_v)
    pltpu.sync_copy(src_hbm.at[pl.ds(e0, _EPW)], src_v)
    pltpu.sync_copy(dst_hbm.at[pl.ds(e0, _EPW)], dst_v)
    pltpu.sync_copy(ew_hbm.at[pl.ds(e0, _EPW)], ew_v)

    zvec = jnp.zeros((16,), jnp.float32)

    @pl.loop(0, _F // 16)
    def _(i):
      acc_v[pl.ds(i * 16, 16)] = zvec

    @pl.loop(0, _NV)
    def _(i):
      b = i * 16
      s16 = src_v[pl.ds(b, 16)]
      d16 = dst_v[pl.ds(b, 16)]
      w16 = ew_v[pl.ds(b, 16)]
      i0 = s16 + s16
      d0 = d16 + d16
      g0 = plsc.load_gather(t01_v, [i0])
      g1 = plsc.load_gather(t01_v, [i0 + 1])
      plsc.addupdate_scatter(acc_v, [d0], g0 * w16)
      plsc.addupdate_scatter(acc_v, [d0 + 1], g1 * w16)

    pltpu.sync_copy(acc_v, accs_sh.at[pl.ds(s * _F, _F)])
    plsc.subcore_barrier()

    def reduce_range(f0, ln):
      pltpu.sync_copy(accs_sh.at[pl.ds(f0, ln)], red_v.at[pl.ds(0, ln)])
      for sp in range(1, _NS):
        pltpu.sync_copy(accs_sh.at[pl.ds(sp * _F + f0, ln)],
                        tmp_v.at[pl.ds(0, ln)])

        @pl.loop(0, ln // 16)
        def _(j):
          red_v[pl.ds(j * 16, 16)] = (red_v[pl.ds(j * 16, 16)] +
                                      tmp_v[pl.ds(j * 16, 16)])
      pltpu.sync_copy(red_v.at[pl.ds(0, ln)], out_hbm.at[pl.ds(c * _F + f0, ln)])

    @pl.when(s < _NS - 1)
    def _():
      reduce_range(s * _FPT, _FPT)

    @pl.when(s == _NS - 1)
    def _():
      reduce_range((_NS - 1) * _FPT, _F - (_NS - 1) * _FPT)

  return pair_kernel


_seg_sum_pair = _make_seg_sum_pair()

_BN = 1000
_GRID = _N // _BN


_DNUMS = (((1,), (1,)), ((), ()))


def _dotT(a, b):
  return lax.dot_general(a, b, dimension_numbers=_DNUMS,
                         preferred_element_type=jnp.float32,
                         precision=lax.Precision.HIGHEST)


def _dense_body(aggp_ref, x_ref, w1rel_ref, w1root_ref, b1_ref, w2rel_ref,
                w2root_ref, t2_ref, r2_ref):
  a = aggp_ref[0] + aggp_ref[1]
  s = _dotT(a, w1rel_ref[...]) + _dotT(x_ref[...], w1root_ref[...])
  s = jnp.maximum(s + b1_ref[...], 0.0)
  t2_ref[...] = _dotT(s, w2rel_ref[...])
  r2_ref[...] = _dotT(s, w2root_ref[...])


def _final_body(aggp_ref, r2_ref, b2_ref, o_ref):
  v = aggp_ref[0] + aggp_ref[1] + r2_ref[...] + b2_ref[...]
  v0 = v[:, 0:1]
  v1 = v[:, 1:2]
  m = jnp.maximum(v0, v1)
  lse = m + jnp.log(jnp.exp(v0 - m) + jnp.exp(v1 - m))
  o_ref[...] = jnp.concatenate([v0 - lse, v1 - lse], axis=1)


@jax.jit
def kernel(x, edge_index, edge_attr, W1_rel, b1_rel, W1_root, W2_rel, b2_rel,
           W2_root):
  src = edge_index[0]
  dst = edge_index[1]

  agg1p = _seg_sum_128(x, src, dst, edge_attr)

  b1 = b1_rel.reshape(1, _H)

  t2, r2 = pl.pallas_call(
      _dense_body,
      grid=(_GRID,),
      in_specs=[
          pl.BlockSpec((_NC, _BN, _D), lambda i: (0, i, 0)),
          pl.BlockSpec((_BN, _D), lambda i: (i, 0)),
          pl.BlockSpec((_H, _D), lambda i: (0, 0)),
          pl.BlockSpec((_H, _D), lambda i: (0, 0)),
          pl.BlockSpec((1, _H), lambda i: (0, 0)),
          pl.BlockSpec((2, _H), lambda i: (0, 0)),
          pl.BlockSpec((2, _H), lambda i: (0, 0)),
      ],
      out_specs=[
          pl.BlockSpec((_BN, 2), lambda i: (i, 0)),
          pl.BlockSpec((_BN, 2), lambda i: (i, 0)),
      ],
      out_shape=[
          jax.ShapeDtypeStruct((_N, 2), jnp.float32),
          jax.ShapeDtypeStruct((_N, 2), jnp.float32),
      ],
  )(agg1p, x, W1_rel, W1_root, b1, W2_rel, W2_root)

  agg2p = _seg_sum_pair(t2.reshape(-1), src, dst, edge_attr)
  agg2p = agg2p.reshape(_NC, _N, 2)

  out = pl.pallas_call(
      _final_body,
      grid=(_GRID,),
      in_specs=[
          pl.BlockSpec((_NC, _BN, 2), lambda i: (0, i, 0)),
          pl.BlockSpec((_BN, 2), lambda i: (i, 0)),
          pl.BlockSpec((1, 2), lambda i: (0, 0)),
      ],
      out_specs=pl.BlockSpec((_BN, 2), lambda i: (i, 0)),
      out_shape=jax.ShapeDtypeStruct((_N, 2), jnp.float32),
  )(agg2p, r2, b2_rel.reshape(1, 2))
  return out


# speedup vs baseline: 1.0285x; 1.0285x over previous
import dataclasses
import functools

import jax
import jax.numpy as jnp
import numpy as np
from jax import lax
from jax.experimental import pallas as pl
from jax.experimental.pallas import tpu as pltpu
from jax.experimental.pallas import tpu_sc as plsc

_N = 10000
_E = 320000
_D = 128
_H = 512
_NC = 2
_NS = 16
_NW = _NC * _NS
_EPW = _E // _NW
_K = 80
_CHUNKS = _EPW // _K
_RPT = 624
_ZR = 16
_SPLAT_IDX = [np.full((16,), i, np.int32) for i in range(16)]


def _make_seg_sum_128():
  mesh = plsc.VectorSubcoreMesh(core_axis_name="c", subcore_axis_name="s")
  jper = _D // 16
  width = _D
  idx_t = [pltpu.VMEM((_K,), jnp.int32)] * 4
  ew_t = [pltpu.VMEM((_K,), jnp.float32)] * 4
  row_t = [pltpu.VMEM((_K, width), jnp.float32)] * 4
  sem_t = [pltpu.SemaphoreType.DMA] * 8

  @functools.partial(
      pl.kernel,
      out_type=jax.ShapeDtypeStruct((_NC, _N, width), jnp.float32),
      mesh=mesh,
      scratch_types=(idx_t + idx_t + ew_t + row_t +
                     [pltpu.VMEM_SHARED((_N, width), jnp.float32)] + sem_t),
      compiler_params=dataclasses.replace(
          pltpu.CompilerParams(), needs_layout_passes=False),
  )
  def seg_kernel(tab_hbm, ei_hbm, ew_hbm, out_hbm,
                 si0, si1, si2, si3, di0, di1, di2, di3, wi0, wi1, wi2, wi3,
                 gb0, gb1, sb0, sb1, acc_sh,
                 gsem0, gsem1, ssem0, ssem1, isem0, isem1, isem2, isem3):
    c = lax.axis_index("c")
    s = lax.axis_index("s")
    wid = c * _NS + s
    e0 = wid * _EPW

    src_ib = (si0, si1, si2, si3)
    dst_ib = (di0, di1, di2, di3)
    ew_ib = (wi0, wi1, wi2, wi3)
    gbufs = (gb0, gb1)
    sbufs = (sb0, sb1)
    gsems = (gsem0, gsem1)
    ssems = (ssem0, ssem1)
    isems = (isem0, isem1, isem2, isem3)

    zvec = jnp.zeros((16,), jnp.float32)

    @pl.loop(0, _ZR)
    def _(i):
      for j in range(jper):
        gb0[i, pl.ds(j * 16, 16)] = zvec

    nz = jnp.where(s == _NS - 1, (_N - (_NS - 1) * _RPT) // _ZR, _RPT // _ZR)

    @pl.loop(0, nz)
    def _(z):
      pltpu.sync_copy(gb0.at[pl.ds(0, _ZR)],
                      acc_sh.at[pl.ds(s * _RPT + z * _ZR, _ZR)])
    plsc.subcore_barrier()

    def fire_idx(cx, b4):
      base = e0 + cx * _K
      pltpu.async_copy(ei_hbm.at[pl.ds(base, _K)], src_ib[b4], isems[b4])
      pltpu.async_copy(ei_hbm.at[pl.ds(_E + base, _K)], dst_ib[b4], isems[b4])
      pltpu.async_copy(ew_hbm.at[pl.ds(base, _K)], ew_ib[b4], isems[b4])

    def wait_idx(cx, b4):
      base = e0 + cx * _K
      pltpu.make_async_copy(ei_hbm.at[pl.ds(base, _K)], src_ib[b4],
                            isems[b4]).wait()
      pltpu.make_async_copy(ei_hbm.at[pl.ds(_E + base, _K)], dst_ib[b4],
                            isems[b4]).wait()
      pltpu.make_async_copy(ew_hbm.at[pl.ds(base, _K)], ew_ib[b4],
                            isems[b4]).wait()

    def fire_gather(cc, b2, b4):
      pltpu.async_copy(tab_hbm.at[src_ib[b4]], gbufs[b2], gsems[b2])

    def wait_gather(b2, b4):
      pltpu.make_async_copy(tab_hbm.at[src_ib[b4]], gbufs[b2],
                            gsems[b2]).wait()

    def fire_scatter(b2, b4):
      pltpu.async_copy(sbufs[b2], acc_sh.at[dst_ib[b4]], ssems[b2], add=True)

    def wait_scatter(b2, b4):
      pltpu.make_async_copy(sbufs[b2], acc_sh.at[dst_ib[b4]],
                            ssems[b2]).wait()

    def slot(cc, b2, b4, steady_tail):
      wait_gather(b2, b4)

      @pl.when(cc >= 2)
      def _():
        wait_scatter(b2, (b4 + 2) % 4)

      if not steady_tail:
        @pl.when(cc + 2 < _CHUNKS)
        def _():
          fire_idx(cc + 2, (b4 + 2) % 4)

      for g in range(_K // 16):
        wvec = ew_ib[b4][pl.ds(g * 16, 16)]
        for e16 in range(16):
          w16 = lax.gather(
              wvec, jnp.full((16, 1), e16, jnp.int32),
              dimension_numbers=lax.GatherDimensionNumbers(
                  offset_dims=(), collapsed_slice_dims=(0,),
                  start_index_map=(0,)),
              slice_sizes=(1,),
              mode=lax.GatherScatterMode.PROMISE_IN_BOUNDS)
          e = g * 16 + e16
          for j in range(jper):
            sbufs[b2][e, pl.ds(j * 16, 16)] = (
                gbufs[b2][e, pl.ds(j * 16, 16)] * w16)

      fire_scatter(b2, b4)
      if not steady_tail:
        @pl.when(cc + 2 < _CHUNKS)
        def _():
          wait_idx(cc + 2, (b4 + 2) % 4)
          fire_gather(cc + 2, b2, (b4 + 2) % 4)

    fire_idx(0, 0)
    fire_idx(1, 1)
    wait_idx(0, 0)
    fire_gather(0, 0, 0)
    wait_idx(1, 1)
    fire_gather(1, 1, 1)

    @pl.loop(0, _CHUNKS - 1, step=4)
    def _(cc):
      slot(cc, 0, 0, False)
      slot(cc + 1, 1, 1, False)
      slot(cc + 2, 0, 2, False)
      slot(cc + 3, 1, 3, False)

    slot(_CHUNKS - 1, 0, 0, True)
    wait_scatter(1, 3)
    wait_scatter(0, 0)

    plsc.subcore_barrier()

    @pl.loop(0, nz)
    def _(z):
      r0 = s * _RPT + z * _ZR
      pltpu.sync_copy(acc_sh.at[pl.ds(r0, _ZR)], out_hbm.at[c, pl.ds(r0, _ZR)])

  return seg_kernel


_seg_sum_128 = _make_seg_sum_128()

_F = 2 * _N
_FPT = 1248
_NV = _EPW // 16


def _make_seg_sum_pair():
  mesh = plsc.VectorSubcoreMesh(core_axis_name="c", subcore_axis_name="s")

  @functools.partial(
      pl.kernel,
      out_type=jax.ShapeDtypeStruct((_NC * _F,), jnp.float32),
      mesh=mesh,
      scratch_types=[
          pltpu.VMEM((_F,), jnp.float32),
          pltpu.VMEM((_EPW,), jnp.int32),
          pltpu.VMEM((_EPW,), jnp.int32),
          pltpu.VMEM((_EPW,), jnp.float32),
          pltpu.VMEM((_F,), jnp.float32),
          pltpu.VMEM((1280,), jnp.float32),
          pltpu.VMEM((1280,), jnp.float32),
          pltpu.VMEM_SHARED((_NS * _F,), jnp.float32),
          pltpu.SemaphoreType.DMA,
      ],
      compiler_params=dataclasses.replace(
          pltpu.CompilerParams(), needs_layout_passes=False),
  )
  def pair_kernel(t01_hbm, ei_hbm, ew_hbm, out_hbm,
                  t01_v, src_v, dst_v, ew_v, acc_v, red_v, tmp_v, accs_sh,
                  sem):
    c = lax.axis_index("c")
    s = lax.axis_index("s")
    wid = c * _NS + s
    e0 = wid * _EPW

    pltpu.sync_copy(t01_hbm, t01_v)
    pltpu.sync_copy(ei_hbm.at[pl.ds(e0, _EPW)], src_v)
    pltpu.sync_copy(ei_hbm.at[pl.ds(_E + e0, _EPW)], dst_v)
    pltpu.sync_copy(ew_hbm.at[pl.ds(e0, _EPW)], ew_v)

    zvec = jnp.zeros((16,), jnp.float32)

    @pl.loop(0, _F // 16)
    def _(i):
      acc_v[pl.ds(i * 16, 16)] = zvec

    @pl.loop(0, _NV)
    def _(i):
      b = i * 16
      s16 = src_v[pl.ds(b, 16)]
      d16 = dst_v[pl.ds(b, 16)]
      w16 = ew_v[pl.ds(b, 16)]
      i0 = s16 + s16
      d0 = d16 + d16
      g0 = plsc.load_gather(t01_v, [i0])
      g1 = plsc.load_gather(t01_v, [i0 + 1])
      plsc.addupdate_scatter(acc_v, [d0], g0 * w16)
      plsc.addupdate_scatter(acc_v, [d0 + 1], g1 * w16)

    pltpu.sync_copy(acc_v, accs_sh.at[pl.ds(s * _F, _F)])
    plsc.subcore_barrier()

    def reduce_range(f0, ln):
      pltpu.sync_copy(accs_sh.at[pl.ds(f0, ln)], red_v.at[pl.ds(0, ln)])
      for sp in range(1, _NS):
        pltpu.sync_copy(accs_sh.at[pl.ds(sp * _F + f0, ln)],
                        tmp_v.at[pl.ds(0, ln)])

        @pl.loop(0, ln // 16)
        def _(j):
          red_v[pl.ds(j * 16, 16)] = (red_v[pl.ds(j * 16, 16)] +
                                      tmp_v[pl.ds(j * 16, 16)])
      pltpu.sync_copy(red_v.at[pl.ds(0, ln)], out_hbm.at[pl.ds(c * _F + f0, ln)])

    @pl.when(s < _NS - 1)
    def _():
      reduce_range(s * _FPT, _FPT)

    @pl.when(s == _NS - 1)
    def _():
      reduce_range((_NS - 1) * _FPT, _F - (_NS - 1) * _FPT)

  return pair_kernel


_seg_sum_pair = _make_seg_sum_pair()

_BN = 2000
_GRID = _N // _BN


_DNUMS = (((1,), (1,)), ((), ()))


def _dotT(a, b):
  return lax.dot_general(a, b, dimension_numbers=_DNUMS,
                         preferred_element_type=jnp.float32,
                         precision=lax.Precision.HIGHEST)


def _dense_body(aggp_ref, x_ref, w1rel_ref, w1root_ref, b1_ref, w2rel_ref,
                w2root_ref, t2_ref, r2_ref):
  a = aggp_ref[0] + aggp_ref[1]
  s = _dotT(a, w1rel_ref[...]) + _dotT(x_ref[...], w1root_ref[...])
  s = jnp.maximum(s + b1_ref[...], 0.0)
  t2_ref[...] = _dotT(s, w2rel_ref[...])
  r2_ref[...] = _dotT(s, w2root_ref[...])


def _final_body(aggp_ref, r2_ref, b2_ref, o_ref):
  v = aggp_ref[0] + aggp_ref[1] + r2_ref[...] + b2_ref[...]
  v0 = v[:, 0:1]
  v1 = v[:, 1:2]
  m = jnp.maximum(v0, v1)
  lse = m + jnp.log(jnp.exp(v0 - m) + jnp.exp(v1 - m))
  o_ref[...] = jnp.concatenate([v0 - lse, v1 - lse], axis=1)


@jax.jit
def kernel(x, edge_index, edge_attr, W1_rel, b1_rel, W1_root, W2_rel, b2_rel,
           W2_root):
  ei = edge_index.reshape(-1)

  agg1p = _seg_sum_128(x, ei, edge_attr)

  b1 = b1_rel.reshape(1, _H)

  t2, r2 = pl.pallas_call(
      _dense_body,
      grid=(_GRID,),
      in_specs=[
          pl.BlockSpec((_NC, _BN, _D), lambda i: (0, i, 0)),
          pl.BlockSpec((_BN, _D), lambda i: (i, 0)),
          pl.BlockSpec((_H, _D), lambda i: (0, 0)),
          pl.BlockSpec((_H, _D), lambda i: (0, 0)),
          pl.BlockSpec((1, _H), lambda i: (0, 0)),
          pl.BlockSpec((2, _H), lambda i: (0, 0)),
          pl.BlockSpec((2, _H), lambda i: (0, 0)),
      ],
      out_specs=[
          pl.BlockSpec((_BN, 2), lambda i: (i, 0)),
          pl.BlockSpec((_BN, 2), lambda i: (i, 0)),
      ],
      out_shape=[
          jax.ShapeDtypeStruct((_N, 2), jnp.float32),
          jax.ShapeDtypeStruct((_N, 2), jnp.float32),
      ],
  )(agg1p, x, W1_rel, W1_root, b1, W2_rel, W2_root)

  agg2p = _seg_sum_pair(t2.reshape(-1), ei, edge_attr)
  agg2p = agg2p.reshape(_NC, _N, 2)

  out = pl.pallas_call(
      _final_body,
      grid=(_GRID,),
      in_specs=[
          pl.BlockSpec((_NC, _BN, 2), lambda i: (0, i, 0)),
          pl.BlockSpec((_BN, 2), lambda i: (i, 0)),
          pl.BlockSpec((1, 2), lambda i: (0, 0)),
      ],
      out_specs=pl.BlockSpec((_BN, 2), lambda i: (i, 0)),
      out_shape=jax.ShapeDtypeStruct((_N, 2), jnp.float32),
  )(agg2p, r2, b2_rel.reshape(1, 2))
  return out
